# Initial kernel scaffold; baseline (speedup 1.0000x reference)
import jax
import jax.numpy as jnp
from jax.experimental import pallas as pl


def kernel(boxes, scores, labels):
    raise NotImplementedError("write your pallas kernel here")


# speedup vs baseline: 108.2006x; 108.2006x over previous
import functools

import jax
import jax.numpy as jnp
from jax import lax
from jax.experimental import pallas as pl
from jax.experimental.pallas import tpu as pltpu
from jax.experimental.pallas import tpu_sc as plsc

N = 20000
NPAD = 20480
NMS_PRE = 4096
NMS_POST = 500
NMS_THRESH = 0.1
SENTINEL = N

SR, SC_ = 256, 128
NSORT = SR * SC_

BLK = 128
NBLK = NMS_PRE // BLK


def _bitonic_stage(key, idx, riota, ciota, k, j):
    if j >= SC_:
        jr = j // SC_
        hi = (riota & jr) != 0
        pk = jnp.where(hi, pltpu.roll(key, jr, 0), pltpu.roll(key, SR - jr, 0))
        pi = jnp.where(hi, pltpu.roll(idx, jr, 0), pltpu.roll(idx, SR - jr, 0))
        jbit = ~hi
    else:
        hi = (ciota & j) != 0
        pk = jnp.where(hi, pltpu.roll(key, j, 1), pltpu.roll(key, SC_ - j, 1))
        pi = jnp.where(hi, pltpu.roll(idx, j, 1), pltpu.roll(idx, SC_ - j, 1))
        jbit = ~hi
    if k >= SC_:
        kbit = (riota & (k // SC_)) == 0
    else:
        kbit = (ciota & k) == 0
    own_first = (key > pk) | ((key == pk) & (idx < pi))
    take = (kbit == jbit) != own_first
    return jnp.where(take, pk, key), jnp.where(take, pi, idx)


def _sort_body(key_ref, idx_ref, oidx_ref):
    key = key_ref[...]
    idx = idx_ref[...]
    riota = lax.broadcasted_iota(jnp.int32, (SR, SC_), 0)
    ciota = lax.broadcasted_iota(jnp.int32, (SR, SC_), 1)
    kk = 2
    while kk <= NSORT:
        j = kk // 2
        while j >= 1:
            key, idx = _bitonic_stage(key, idx, riota, ciota, kk, j)
            j //= 2
        kk *= 2
    oidx_ref[...] = idx


_sort_call = pl.pallas_call(
    _sort_body,
    out_shape=jax.ShapeDtypeStruct((SR, SC_), jnp.int32),
)


_NCORES = 2
_NSUB = 16
_NW = _NCORES * _NSUB


def _make_row_gather(nrows):
    per_w = nrows // _NW
    mesh = plsc.VectorSubcoreMesh(core_axis_name="c", subcore_axis_name="s", num_cores=_NCORES, num_subcores=_NSUB)

    @functools.partial(
        pl.kernel,
        out_type=jax.ShapeDtypeStruct((nrows, 8), jnp.float32),
        mesh=mesh,
        compiler_params=pltpu.CompilerParams(use_tc_tiling_on_sc=False, needs_layout_passes=False),
        scratch_types=[
            pltpu.VMEM((per_w,), jnp.int32),
            pltpu.VMEM((per_w, 8), jnp.float32),
            pltpu.SemaphoreType.DMA,
        ],
    )
    def gather_rows(table_hbm, idx_hbm, out_hbm, idx_v, rows_v, sem):
        wid = lax.axis_index("s") * _NCORES + lax.axis_index("c")
        base = wid * per_w
        pltpu.sync_copy(idx_hbm.at[pl.ds(base, per_w)], idx_v)
        pltpu.async_copy(table_hbm.at[idx_v], rows_v, sem).wait()
        pltpu.sync_copy(rows_v, out_hbm.at[pl.ds(base, per_w)])

    return gather_rows


_gather_cand = functools.lru_cache(None)(_make_row_gather)


GCH = 2048
GCC = 1024


def _nms_body(candc_ref, btab_ref, keep_ref, alive_ref):
    candc = candc_ref[...]
    gcols = []
    for cc in range(NMS_PRE // GCC):
        cslice = candc[:, cc * GCC:(cc + 1) * GCC]
        acc = jnp.zeros((8, GCC), jnp.float32)
        for ch in range(NPAD // GCH):
            rows = (lax.broadcasted_iota(jnp.int32, (GCH, 1), 0)
                    + (ch * GCH)).astype(jnp.float32)
            oh = (rows == cslice).astype(jnp.float32)
            acc = acc + jnp.dot(btab_ref[:, ch * GCH:(ch + 1) * GCH], oh,
                                precision=lax.Precision.HIGHEST,
                                preferred_element_type=jnp.float32)
        gcols.append(acc)
    gt = jnp.concatenate(gcols, axis=1)
    g = jnp.transpose(gt)

    cxc, cyc = gt[0:1, :], gt[1:2, :]
    wc, hc = gt[3:4, :], gt[4:5, :]
    cxr, cyr = g[:, 0:1], g[:, 1:2]
    wr, hr = g[:, 3:4], g[:, 4:5]

    alive_ref[...] = jnp.ones((1, NMS_PRE), jnp.float32)
    colpos = lax.broadcasted_iota(jnp.int32, (1, NMS_PRE), 1)

    xmin_c = cxc - wc * 0.5
    xmax_c = cxc + wc * 0.5
    ymin_c = cyc - hc * 0.5
    ymax_c = cyc + hc * 0.5
    area_c = wc * hc

    for b in range(NBLK):
        base = b * BLK
        rs = slice(base, base + BLK)
        cxb, wb = cxr[rs, :], wr[rs, :]
        cyb, hb = cyr[rs, :], hr[rs, :]
        xmin_r = cxb - wb * 0.5
        xmax_r = cxb + wb * 0.5
        ymin_r = cyb - hb * 0.5
        ymax_r = cyb + hb * 0.5
        area_r = wb * hb

        cs = slice(base, NMS_PRE)
        iw = jnp.clip(jnp.minimum(xmax_r, xmax_c[:, cs])
                      - jnp.maximum(xmin_r, xmin_c[:, cs]), 0.0, None)
        ih = jnp.clip(jnp.minimum(ymax_r, ymax_c[:, cs])
                      - jnp.maximum(ymin_r, ymin_c[:, cs]), 0.0, None)
        inter = iw * ih
        union = area_r + area_c[:, cs] - inter
        iou = inter / jnp.clip(union, 1e-08, None)
        rowpos = lax.broadcasted_iota(jnp.int32, (BLK, 1), 0)
        supf = ((iou > NMS_THRESH) & (colpos[:, cs] - base > rowpos)
                ).astype(jnp.float32)

        sup_blk = supf[:, :BLK]
        alive0 = alive_ref[:, base:base + BLK]

        def jacobi_cond(carry):
            _, changed, t = carry
            return changed & (t < BLK)

        def jacobi_body(carry):
            keep, _, t = carry
            s = jnp.dot(keep, sup_blk, preferred_element_type=jnp.float32)
            keep_new = alive0 * (s < 0.5).astype(jnp.float32)
            changed = jnp.any(keep_new != keep)
            return keep_new, changed, t + 1

        alive_blk, _, _ = lax.while_loop(
            jacobi_cond, jacobi_body, (alive0, jnp.bool_(True), jnp.int32(0)))
        alive_ref[:, base:base + BLK] = alive_blk

        s = jnp.dot(alive_blk, supf, preferred_element_type=jnp.float32)
        alive_ref[:, base:] = alive_ref[:, base:] * (s < 0.5).astype(jnp.float32)

    keep_ref[...] = alive_ref[...]


_nms_call = pl.pallas_call(
    _nms_body,
    out_shape=jax.ShapeDtypeStruct((1, NMS_PRE), jnp.float32),
    scratch_shapes=[
        pltpu.VMEM((1, NMS_PRE), jnp.float32),
    ],
)


def _make_finalize():
    nrows = 512
    per_w = nrows // _NW
    mesh = plsc.VectorSubcoreMesh(core_axis_name="c", subcore_axis_name="s", num_cores=_NCORES, num_subcores=_NSUB)

    @functools.partial(
        pl.kernel,
        out_type=(jax.ShapeDtypeStruct((nrows, 8), jnp.float32),
                  jax.ShapeDtypeStruct((nrows,), jnp.float32),
                  jax.ShapeDtypeStruct((nrows,), jnp.float32)),
        mesh=mesh,
        compiler_params=pltpu.CompilerParams(use_tc_tiling_on_sc=False, needs_layout_passes=False),
        scratch_types=[
            pltpu.VMEM((NMS_PRE,), jnp.float32),
            pltpu.VMEM((NMS_PRE,), jnp.int32),
            pltpu.VMEM((nrows,), jnp.int32),
            pltpu.VMEM((per_w,), jnp.int32),
            pltpu.VMEM((per_w, 8), jnp.float32),
            pltpu.VMEM((per_w,), jnp.float32),
            pltpu.VMEM((per_w,), jnp.float32),
            pltpu.SemaphoreType.DMA,
        ],
    )
    def finalize(keep_hbm, cand_hbm, table_hbm, sc_hbm, lb_hbm,
                 obox_hbm, osc_hbm, olb_hbm,
                 keep_v, cand_v, slots_v, idx_v, rows_v, s_v, l_v, sem):
        wid = lax.axis_index("s") * _NCORES + lax.axis_index("c")
        base = wid * per_w
        pltpu.sync_copy(keep_hbm, keep_v)
        pltpu.sync_copy(cand_hbm, cand_v)
        sent = jnp.full((16,), SENTINEL, jnp.int32)
        for i in range(nrows // 16):
            slots_v[pl.ds(i * 16, 16)] = sent

        def body(i, carry):
            kv = keep_v[pl.ds(i * 16, 16)]
            incl = plsc.cumsum(kv)
            ranks = (incl + carry).astype(jnp.int32) - 1
            valid = (kv > 0.5) & (ranks < NMS_POST)
            slot = jnp.where(valid, ranks, 0)
            cv = cand_v[pl.ds(i * 16, 16)]
            plsc.store_scatter(slots_v, [slot], cv, mask=valid)
            return carry + jnp.sum(kv)

        lax.fori_loop(0, NMS_PRE // 16, body, jnp.float32(0.0))

        idx_v[...] = slots_v[pl.ds(base, per_w)]
        pltpu.async_copy(table_hbm.at[idx_v], rows_v, sem).wait()
        pltpu.async_copy(sc_hbm.at[idx_v], s_v, sem).wait()
        pltpu.async_copy(lb_hbm.at[idx_v], l_v, sem).wait()
        pltpu.sync_copy(rows_v, obox_hbm.at[pl.ds(base, per_w)])
        pltpu.sync_copy(s_v, osc_hbm.at[pl.ds(base, per_w)])
        pltpu.sync_copy(l_v, olb_hbm.at[pl.ds(base, per_w)])

    return finalize


_finalize = functools.lru_cache(None)(_make_finalize)


def kernel(boxes, scores, labels):
    keys = lax.bitcast_convert_type(scores, jnp.int32)
    keys = jnp.concatenate([keys, jnp.full((NSORT - N,), -1, jnp.int32)])
    idx = jnp.arange(NSORT, dtype=jnp.int32)
    sorted_idx = _sort_call(keys.reshape(SR, SC_), idx.reshape(SR, SC_))
    cand = sorted_idx.reshape(-1)[:NMS_PRE]

    boxes8 = jnp.zeros((NPAD, 8), jnp.float32).at[:N, :7].set(boxes)
    scores_p = jnp.zeros((NPAD,), jnp.float32).at[:N].set(scores)
    labels_p = jnp.zeros((NPAD,), jnp.float32).at[:N].set(
        labels.astype(jnp.float32))

    keep = _nms_call(cand.astype(jnp.float32)[None, :], boxes8.T)

    fb, fs, fl = _finalize()(keep.reshape(NMS_PRE), cand,
                             boxes8, scores_p, labels_p)
    return (fl[:NMS_POST][None, :], fs[:NMS_POST][None, :],
            fb[:NMS_POST, :7][None, :, :])


# speedup vs baseline: 202.3886x; 1.8705x over previous
import functools

import jax
import jax.numpy as jnp
from jax import lax
from jax.experimental import pallas as pl
from jax.experimental.pallas import tpu as pltpu
from jax.experimental.pallas import tpu_sc as plsc

N = 20000
NPAD = 20008
NMS_PRE = 4096
NMS_POST = 500
NMS_THRESH = 0.1
SENTINEL = N

SR, SC_ = 256, 128
NSORT = SR * SC_

BLK = 128
NBLK = NMS_PRE // BLK


def _roll_pair(v, hi, a, n, shift):
    return jnp.where(hi, pltpu.roll(v, shift, a), pltpu.roll(v, n - shift, a))


def _bitonic_stage(key, idx, vals, riota, ciota, k, j):
    if j >= SC_:
        jr = j // SC_
        hi = (riota & jr) != 0
        a, n, sh = 0, SR, jr
    else:
        hi = (ciota & j) != 0
        a, n, sh = 1, SC_, j
    pk = _roll_pair(key, hi, a, n, sh)
    pi = _roll_pair(idx, hi, a, n, sh)
    pv = [_roll_pair(v, hi, a, n, sh) for v in vals]
    jbit = ~hi
    if k >= SC_:
        kbit = (riota & (k // SC_)) == 0
    else:
        kbit = (ciota & k) == 0
    own_first = (key > pk) | ((key == pk) & (idx < pi))
    take = (kbit == jbit) != own_first
    return (jnp.where(take, pk, key), jnp.where(take, pi, idx),
            [jnp.where(take, b, a_) for a_, b in zip(vals, pv)])


def _sort_body(key_ref, idx_ref, c0_ref, c1_ref, c2_ref, c3_ref,
               oidx_ref, o0_ref, o1_ref, o2_ref, o3_ref):
    key = key_ref[...]
    idx = idx_ref[...]
    vals = [c0_ref[...], c1_ref[...], c2_ref[...], c3_ref[...]]
    riota = lax.broadcasted_iota(jnp.int32, (SR, SC_), 0)
    ciota = lax.broadcasted_iota(jnp.int32, (SR, SC_), 1)
    kk = 2
    while kk <= NSORT:
        j = kk // 2
        while j >= 1:
            key, idx, vals = _bitonic_stage(key, idx, vals, riota, ciota, kk, j)
            j //= 2
        kk *= 2
    oidx_ref[...] = idx
    o0_ref[...] = vals[0]
    o1_ref[...] = vals[1]
    o2_ref[...] = vals[2]
    o3_ref[...] = vals[3]


_sort_call = pl.pallas_call(
    _sort_body,
    out_shape=(jax.ShapeDtypeStruct((SR, SC_), jnp.int32),)
    + (jax.ShapeDtypeStruct((SR, SC_), jnp.float32),) * 4,
)


_NCORES = 2
_NSUB = 16
_NW = _NCORES * _NSUB


def _make_row_gather(nrows):
    per_w = nrows // _NW
    mesh = plsc.VectorSubcoreMesh(core_axis_name="c", subcore_axis_name="s", num_cores=_NCORES, num_subcores=_NSUB)

    @functools.partial(
        pl.kernel,
        out_type=jax.ShapeDtypeStruct((nrows, 8), jnp.float32),
        mesh=mesh,
        compiler_params=pltpu.CompilerParams(use_tc_tiling_on_sc=False, needs_layout_passes=False),
        scratch_types=[
            pltpu.VMEM((per_w,), jnp.int32),
            pltpu.VMEM((per_w, 8), jnp.float32),
            pltpu.SemaphoreType.DMA,
        ],
    )
    def gather_rows(table_hbm, idx_hbm, out_hbm, idx_v, rows_v, sem):
        wid = lax.axis_index("s") * _NCORES + lax.axis_index("c")
        base = wid * per_w
        pltpu.sync_copy(idx_hbm.at[pl.ds(base, per_w)], idx_v)
        pltpu.async_copy(table_hbm.at[idx_v], rows_v, sem).wait()
        pltpu.sync_copy(rows_v, out_hbm.at[pl.ds(base, per_w)])

    return gather_rows


_gather_cand = functools.lru_cache(None)(_make_row_gather)


def _nms_body(cxr, cyr, wr, hr, cxc, cyc, wc, hc, keep_ref, alive_ref):
    alive_ref[...] = jnp.ones((1, NMS_PRE), jnp.float32)
    colpos = lax.broadcasted_iota(jnp.int32, (1, NMS_PRE), 1)

    xmin_c = cxc[...] - wc[...] * 0.5
    xmax_c = cxc[...] + wc[...] * 0.5
    ymin_c = cyc[...] - hc[...] * 0.5
    ymax_c = cyc[...] + hc[...] * 0.5
    area_c = wc[...] * hc[...]

    for b in range(NBLK):
        base = b * BLK
        rs = pl.ds(base, BLK)
        cxb, wb = cxr[rs, :], wr[rs, :]
        cyb, hb = cyr[rs, :], hr[rs, :]
        xmin_r = cxb - wb * 0.5
        xmax_r = cxb + wb * 0.5
        ymin_r = cyb - hb * 0.5
        ymax_r = cyb + hb * 0.5
        area_r = wb * hb

        cs = slice(base, NMS_PRE)
        iw = jnp.clip(jnp.minimum(xmax_r, xmax_c[:, cs])
                      - jnp.maximum(xmin_r, xmin_c[:, cs]), 0.0, None)
        ih = jnp.clip(jnp.minimum(ymax_r, ymax_c[:, cs])
                      - jnp.maximum(ymin_r, ymin_c[:, cs]), 0.0, None)
        inter = iw * ih
        union = area_r + area_c[:, cs] - inter
        iou = inter / jnp.clip(union, 1e-08, None)
        rowpos = lax.broadcasted_iota(jnp.int32, (BLK, 1), 0)
        supf = ((iou > NMS_THRESH) & (colpos[:, cs] - base > rowpos)
                ).astype(jnp.float32)

        sup_blk = supf[:, :BLK]
        alive0 = alive_ref[:, base:base + BLK]

        def jacobi_cond(carry):
            _, changed, t = carry
            return changed & (t < BLK)

        def jacobi_body(carry):
            keep, _, t = carry
            s = jnp.dot(keep, sup_blk, preferred_element_type=jnp.float32)
            keep_new = alive0 * (s < 0.5).astype(jnp.float32)
            changed = jnp.any(keep_new != keep)
            return keep_new, changed, t + 1

        alive_blk, _, _ = lax.while_loop(
            jacobi_cond, jacobi_body, (alive0, jnp.bool_(True), jnp.int32(0)))
        alive_ref[:, base:base + BLK] = alive_blk

        s = jnp.dot(alive_blk, supf, preferred_element_type=jnp.float32)
        alive_ref[:, base:] = alive_ref[:, base:] * (s < 0.5).astype(jnp.float32)

    keep_ref[...] = alive_ref[...]


_nms_call = pl.pallas_call(
    _nms_body,
    out_shape=jax.ShapeDtypeStruct((1, NMS_PRE), jnp.float32),
    scratch_shapes=[
        pltpu.VMEM((1, NMS_PRE), jnp.float32),
    ],
)


def _make_finalize():
    nrows = 512
    per_w = nrows // _NW
    mesh = plsc.VectorSubcoreMesh(core_axis_name="c", subcore_axis_name="s", num_cores=_NCORES, num_subcores=_NSUB)

    @functools.partial(
        pl.kernel,
        out_type=(jax.ShapeDtypeStruct((nrows, 8), jnp.float32),
                  jax.ShapeDtypeStruct((nrows,), jnp.float32),
                  jax.ShapeDtypeStruct((nrows,), jnp.float32)),
        mesh=mesh,
        compiler_params=pltpu.CompilerParams(use_tc_tiling_on_sc=False, needs_layout_passes=False),
        scratch_types=[
            pltpu.VMEM((NMS_PRE,), jnp.float32),
            pltpu.VMEM((NMS_PRE,), jnp.int32),
            pltpu.VMEM((nrows,), jnp.int32),
            pltpu.VMEM((per_w,), jnp.int32),
            pltpu.VMEM((per_w, 8), jnp.float32),
            pltpu.VMEM((per_w,), jnp.float32),
            pltpu.VMEM((per_w,), jnp.float32),
            pltpu.SemaphoreType.DMA,
        ],
    )
    def finalize(keep_hbm, cand_hbm, table_hbm, sc_hbm, lb_hbm,
                 obox_hbm, osc_hbm, olb_hbm,
                 keep_v, cand_v, slots_v, idx_v, rows_v, s_v, l_v, sem):
        wid = lax.axis_index("s") * _NCORES + lax.axis_index("c")
        base = wid * per_w
        pltpu.sync_copy(keep_hbm, keep_v)
        pltpu.sync_copy(cand_hbm, cand_v)
        sent = jnp.full((16,), SENTINEL, jnp.int32)
        for i in range(nrows // 16):
            slots_v[pl.ds(i * 16, 16)] = sent

        def body(i, carry):
            kv = keep_v[pl.ds(i * 16, 16)]
            incl = plsc.cumsum(kv)
            ranks = (incl + carry).astype(jnp.int32) - 1
            valid = (kv > 0.5) & (ranks < NMS_POST)
            slot = jnp.where(valid, ranks, 0)
            cv = cand_v[pl.ds(i * 16, 16)]
            plsc.store_scatter(slots_v, [slot], cv, mask=valid)
            return carry + jnp.sum(kv)

        lax.fori_loop(0, NMS_PRE // 16, body, jnp.float32(0.0))

        idx_v[...] = slots_v[pl.ds(base, per_w)]
        pltpu.async_copy(table_hbm.at[idx_v], rows_v, sem).wait()
        pltpu.async_copy(sc_hbm.at[idx_v], s_v, sem).wait()
        pltpu.async_copy(lb_hbm.at[idx_v], l_v, sem).wait()
        pltpu.sync_copy(rows_v, obox_hbm.at[pl.ds(base, per_w)])
        pltpu.sync_copy(s_v, osc_hbm.at[pl.ds(base, per_w)])
        pltpu.sync_copy(l_v, olb_hbm.at[pl.ds(base, per_w)])

    return finalize


_finalize = functools.lru_cache(None)(_make_finalize)


def kernel(boxes, scores, labels):
    keys = lax.bitcast_convert_type(scores, jnp.int32)
    keys = jnp.concatenate([keys, jnp.full((NSORT - N,), -1, jnp.int32)])
    idx = jnp.arange(NSORT, dtype=jnp.int32)
    padc = jnp.zeros((NSORT - N,), jnp.float32)
    cols = [jnp.concatenate([boxes[:, c], padc]).reshape(SR, SC_)
            for c in (0, 1, 3, 4)]
    sorted_idx, cx, cy, w, h = _sort_call(
        keys.reshape(SR, SC_), idx.reshape(SR, SC_), *cols)
    cand = sorted_idx.reshape(-1)[:NMS_PRE]
    cx = cx.reshape(-1)[:NMS_PRE]
    cy = cy.reshape(-1)[:NMS_PRE]
    w = w.reshape(-1)[:NMS_PRE]
    h = h.reshape(-1)[:NMS_PRE]

    boxes8 = jnp.zeros((NPAD, 8), jnp.float32).at[:N, :7].set(boxes)
    scores_p = jnp.zeros((NPAD,), jnp.float32).at[:N].set(scores)
    labels_p = jnp.zeros((NPAD,), jnp.float32).at[:N].set(
        labels.astype(jnp.float32))

    keep = _nms_call(cx[:, None], cy[:, None], w[:, None], h[:, None],
                     cx[None, :], cy[None, :], w[None, :], h[None, :])

    fb, fs, fl = _finalize()(keep.reshape(NMS_PRE), cand,
                             boxes8, scores_p, labels_p)
    return (fl[:NMS_POST][None, :], fs[:NMS_POST][None, :],
            fb[:NMS_POST, :7][None, :, :])


# speedup vs baseline: 251.9013x; 1.2446x over previous
import functools

import jax
import jax.numpy as jnp
from jax import lax
from jax.experimental import pallas as pl
from jax.experimental.pallas import tpu as pltpu
from jax.experimental.pallas import tpu_sc as plsc

N = 20000
NPAD = 20008
NMS_PRE = 4096
NMS_POST = 500
NMS_THRESH = 0.1
SENTINEL = N

SR, SC_ = 256, 128
NSORT = SR * SC_

BLK = 256
NBLK = NMS_PRE // BLK


def _bitonic_stage(key, idx, riota, ciota, k, j):
    if j >= SC_:
        jr = j // SC_
        hi = (riota & jr) != 0
        pk = jnp.where(hi, pltpu.roll(key, jr, 0), pltpu.roll(key, SR - jr, 0))
        pi = jnp.where(hi, pltpu.roll(idx, jr, 0), pltpu.roll(idx, SR - jr, 0))
        jbit = ~hi
    else:
        hi = (ciota & j) != 0
        pk = jnp.where(hi, pltpu.roll(key, j, 1), pltpu.roll(key, SC_ - j, 1))
        pi = jnp.where(hi, pltpu.roll(idx, j, 1), pltpu.roll(idx, SC_ - j, 1))
        jbit = ~hi
    if k >= SC_:
        kbit = (riota & (k // SC_)) == 0
    else:
        kbit = (ciota & k) == 0
    own_first = (key > pk) | ((key == pk) & (idx < pi))
    take = (kbit == jbit) != own_first
    return jnp.where(take, pk, key), jnp.where(take, pi, idx)


def _sort_body(key_ref, idx_ref, oidx_ref):
    key = key_ref[...]
    idx = idx_ref[...]
    riota = lax.broadcasted_iota(jnp.int32, (SR, SC_), 0)
    ciota = lax.broadcasted_iota(jnp.int32, (SR, SC_), 1)
    kk = 2
    while kk <= NSORT:
        j = kk // 2
        while j >= 1:
            key, idx = _bitonic_stage(key, idx, riota, ciota, kk, j)
            j //= 2
        kk *= 2
    oidx_ref[...] = idx


_sort_call = pl.pallas_call(
    _sort_body,
    out_shape=jax.ShapeDtypeStruct((SR, SC_), jnp.int32),
)


_NCORES = 2
_NSUB = 16
_NW = _NCORES * _NSUB


def _make_row_gather(nrows):
    per_w = nrows // _NW
    mesh = plsc.VectorSubcoreMesh(core_axis_name="c", subcore_axis_name="s", num_cores=_NCORES, num_subcores=_NSUB)

    @functools.partial(
        pl.kernel,
        out_type=jax.ShapeDtypeStruct((nrows, 8), jnp.float32),
        mesh=mesh,
        compiler_params=pltpu.CompilerParams(use_tc_tiling_on_sc=False, needs_layout_passes=False),
        scratch_types=[
            pltpu.VMEM((per_w,), jnp.int32),
            pltpu.VMEM((per_w, 8), jnp.float32),
            pltpu.SemaphoreType.DMA,
        ],
    )
    def gather_rows(table_hbm, idx_hbm, out_hbm, idx_v, rows_v, sem):
        wid = lax.axis_index("s") * _NCORES + lax.axis_index("c")
        base = wid * per_w
        pltpu.sync_copy(idx_hbm.at[pl.ds(base, per_w)], idx_v)
        pltpu.async_copy(table_hbm.at[idx_v], rows_v, sem).wait()
        pltpu.sync_copy(rows_v, out_hbm.at[pl.ds(base, per_w)])

    return gather_rows


_gather_cand = functools.lru_cache(None)(_make_row_gather)


def _nms_body(cxr, cyr, wr, hr, cxc, cyc, wc, hc, keep_ref, alive_ref):
    alive_ref[...] = jnp.ones((1, NMS_PRE), jnp.float32)
    colpos = lax.broadcasted_iota(jnp.int32, (1, NMS_PRE), 1)

    xmin_c = cxc[...] - wc[...] * 0.5
    xmax_c = cxc[...] + wc[...] * 0.5
    ymin_c = cyc[...] - hc[...] * 0.5
    ymax_c = cyc[...] + hc[...] * 0.5
    area_c = wc[...] * hc[...]

    for b in range(NBLK):
        base = b * BLK
        rs = pl.ds(base, BLK)
        cxb, wb = cxr[rs, :], wr[rs, :]
        cyb, hb = cyr[rs, :], hr[rs, :]
        xmin_r = cxb - wb * 0.5
        xmax_r = cxb + wb * 0.5
        ymin_r = cyb - hb * 0.5
        ymax_r = cyb + hb * 0.5
        area_r = wb * hb

        cs = slice(base, NMS_PRE)
        iw = jnp.clip(jnp.minimum(xmax_r, xmax_c[:, cs])
                      - jnp.maximum(xmin_r, xmin_c[:, cs]), 0.0, None)
        ih = jnp.clip(jnp.minimum(ymax_r, ymax_c[:, cs])
                      - jnp.maximum(ymin_r, ymin_c[:, cs]), 0.0, None)
        inter = iw * ih
        union = area_r + area_c[:, cs] - inter
        iou = inter / jnp.clip(union, 1e-08, None)
        rowpos = lax.broadcasted_iota(jnp.int32, (BLK, 1), 0)
        supf = ((iou > NMS_THRESH) & (colpos[:, cs] - base > rowpos)
                ).astype(jnp.float32)

        sup_blk = supf[:, :BLK]
        alive0 = alive_ref[:, base:base + BLK]

        def jacobi_cond(carry):
            _, changed, t = carry
            return changed & (t < BLK)

        def jacobi_body(carry):
            keep, _, t = carry
            s = jnp.dot(keep, sup_blk, preferred_element_type=jnp.float32)
            keep_new = alive0 * (s < 0.5).astype(jnp.float32)
            changed = jnp.any(keep_new != keep)
            return keep_new, changed, t + 1

        alive_blk, _, _ = lax.while_loop(
            jacobi_cond, jacobi_body, (alive0, jnp.bool_(True), jnp.int32(0)))
        alive_ref[:, base:base + BLK] = alive_blk

        s = jnp.dot(alive_blk, supf, preferred_element_type=jnp.float32)
        alive_ref[:, base:] = alive_ref[:, base:] * (s < 0.5).astype(jnp.float32)

    keep_ref[...] = alive_ref[...]


_nms_call = pl.pallas_call(
    _nms_body,
    out_shape=jax.ShapeDtypeStruct((1, NMS_PRE), jnp.float32),
    scratch_shapes=[
        pltpu.VMEM((1, NMS_PRE), jnp.float32),
    ],
)


def _make_finalize():
    nrows = 512
    per_w = nrows // _NW
    mesh = plsc.VectorSubcoreMesh(core_axis_name="c", subcore_axis_name="s", num_cores=_NCORES, num_subcores=_NSUB)

    @functools.partial(
        pl.kernel,
        out_type=(jax.ShapeDtypeStruct((nrows, 8), jnp.float32),
                  jax.ShapeDtypeStruct((nrows,), jnp.float32),
                  jax.ShapeDtypeStruct((nrows,), jnp.float32)),
        mesh=mesh,
        compiler_params=pltpu.CompilerParams(use_tc_tiling_on_sc=False, needs_layout_passes=False),
        scratch_types=[
            pltpu.VMEM((NMS_PRE,), jnp.float32),
            pltpu.VMEM((NMS_PRE,), jnp.int32),
            pltpu.VMEM((nrows,), jnp.int32),
            pltpu.VMEM((per_w,), jnp.int32),
            pltpu.VMEM((per_w, 8), jnp.float32),
            pltpu.VMEM((per_w,), jnp.float32),
            pltpu.VMEM((per_w,), jnp.float32),
            pltpu.SemaphoreType.DMA,
        ],
    )
    def finalize(keep_hbm, cand_hbm, table_hbm, sc_hbm, lb_hbm,
                 obox_hbm, osc_hbm, olb_hbm,
                 keep_v, cand_v, slots_v, idx_v, rows_v, s_v, l_v, sem):
        wid = lax.axis_index("s") * _NCORES + lax.axis_index("c")
        base = wid * per_w
        pltpu.sync_copy(keep_hbm, keep_v)
        pltpu.sync_copy(cand_hbm, cand_v)
        sent = jnp.full((16,), SENTINEL, jnp.int32)
        for i in range(nrows // 16):
            slots_v[pl.ds(i * 16, 16)] = sent

        def body(i, carry):
            kv = keep_v[pl.ds(i * 16, 16)]
            incl = plsc.cumsum(kv)
            ranks = (incl + carry).astype(jnp.int32) - 1
            valid = (kv > 0.5) & (ranks < NMS_POST)
            slot = jnp.where(valid, ranks, 0)
            cv = cand_v[pl.ds(i * 16, 16)]
            plsc.store_scatter(slots_v, [slot], cv, mask=valid)
            return carry + jnp.sum(kv)

        lax.fori_loop(0, NMS_PRE // 16, body, jnp.float32(0.0))

        idx_v[...] = slots_v[pl.ds(base, per_w)]
        pltpu.async_copy(table_hbm.at[idx_v], rows_v, sem).wait()
        pltpu.async_copy(sc_hbm.at[idx_v], s_v, sem).wait()
        pltpu.async_copy(lb_hbm.at[idx_v], l_v, sem).wait()
        pltpu.sync_copy(rows_v, obox_hbm.at[pl.ds(base, per_w)])
        pltpu.sync_copy(s_v, osc_hbm.at[pl.ds(base, per_w)])
        pltpu.sync_copy(l_v, olb_hbm.at[pl.ds(base, per_w)])

    return finalize


_finalize = functools.lru_cache(None)(_make_finalize)


def kernel(boxes, scores, labels):
    keys = lax.bitcast_convert_type(scores, jnp.int32)
    keys = jnp.concatenate([keys, jnp.full((NSORT - N,), -1, jnp.int32)])
    idx = jnp.arange(NSORT, dtype=jnp.int32)
    sorted_idx = _sort_call(keys.reshape(SR, SC_), idx.reshape(SR, SC_))
    cand = sorted_idx.reshape(-1)[:NMS_PRE]

    boxes8 = jnp.zeros((NPAD, 8), jnp.float32).at[:N, :7].set(boxes)
    scores_p = jnp.zeros((NPAD,), jnp.float32).at[:N].set(scores)
    labels_p = jnp.zeros((NPAD,), jnp.float32).at[:N].set(
        labels.astype(jnp.float32))

    cb = _gather_cand(NMS_PRE)(boxes8, cand)
    cx, cy, w, h = cb[:, 0], cb[:, 1], cb[:, 3], cb[:, 4]

    keep = _nms_call(cx[:, None], cy[:, None], w[:, None], h[:, None],
                     cx[None, :], cy[None, :], w[None, :], h[None, :])

    fb, fs, fl = _finalize()(keep.reshape(NMS_PRE), cand,
                             boxes8, scores_p, labels_p)
    return (fl[:NMS_POST][None, :], fs[:NMS_POST][None, :],
            fb[:NMS_POST, :7][None, :, :])


# speedup vs baseline: 260.9459x; 1.0359x over previous
import functools

import jax
import jax.numpy as jnp
from jax import lax
from jax.experimental import pallas as pl
from jax.experimental.pallas import tpu as pltpu
from jax.experimental.pallas import tpu_sc as plsc

N = 20000
NPAD = 20008
NMS_PRE = 4096
NMS_POST = 500
NMS_THRESH = 0.1
SENTINEL = N

SR, SC_ = 256, 128
NSORT = SR * SC_

BLK = 512
NBLK = NMS_PRE // BLK


def _bitonic_stage(key, idx, riota, ciota, k, j):
    if j >= SC_:
        jr = j // SC_
        hi = (riota & jr) != 0
        pk = jnp.where(hi, pltpu.roll(key, jr, 0), pltpu.roll(key, SR - jr, 0))
        pi = jnp.where(hi, pltpu.roll(idx, jr, 0), pltpu.roll(idx, SR - jr, 0))
        jbit = ~hi
    else:
        hi = (ciota & j) != 0
        pk = jnp.where(hi, pltpu.roll(key, j, 1), pltpu.roll(key, SC_ - j, 1))
        pi = jnp.where(hi, pltpu.roll(idx, j, 1), pltpu.roll(idx, SC_ - j, 1))
        jbit = ~hi
    if k >= SC_:
        kbit = (riota & (k // SC_)) == 0
    else:
        kbit = (ciota & k) == 0
    own_first = (key > pk) | ((key == pk) & (idx < pi))
    take = (kbit == jbit) != own_first
    return jnp.where(take, pk, key), jnp.where(take, pi, idx)


def _sort_body(key_ref, idx_ref, oidx_ref):
    key = key_ref[...]
    idx = idx_ref[...]
    riota = lax.broadcasted_iota(jnp.int32, (SR, SC_), 0)
    ciota = lax.broadcasted_iota(jnp.int32, (SR, SC_), 1)
    kk = 2
    while kk <= NSORT:
        j = kk // 2
        while j >= 1:
            key, idx = _bitonic_stage(key, idx, riota, ciota, kk, j)
            j //= 2
        kk *= 2
    oidx_ref[...] = idx


_sort_call = pl.pallas_call(
    _sort_body,
    out_shape=jax.ShapeDtypeStruct((SR, SC_), jnp.int32),
)


_NCORES = 2
_NSUB = 16
_NW = _NCORES * _NSUB


def _make_row_gather(nrows):
    per_w = nrows // _NW
    mesh = plsc.VectorSubcoreMesh(core_axis_name="c", subcore_axis_name="s", num_cores=_NCORES, num_subcores=_NSUB)

    @functools.partial(
        pl.kernel,
        out_type=jax.ShapeDtypeStruct((nrows, 8), jnp.float32),
        mesh=mesh,
        compiler_params=pltpu.CompilerParams(use_tc_tiling_on_sc=False, needs_layout_passes=False),
        scratch_types=[
            pltpu.VMEM((per_w,), jnp.int32),
            pltpu.VMEM((per_w, 8), jnp.float32),
            pltpu.SemaphoreType.DMA,
        ],
    )
    def gather_rows(table_hbm, idx_hbm, out_hbm, idx_v, rows_v, sem):
        wid = lax.axis_index("s") * _NCORES + lax.axis_index("c")
        base = wid * per_w
        pltpu.sync_copy(idx_hbm.at[pl.ds(base, per_w)], idx_v)
        pltpu.async_copy(table_hbm.at[idx_v], rows_v, sem).wait()
        pltpu.sync_copy(rows_v, out_hbm.at[pl.ds(base, per_w)])

    return gather_rows


_gather_cand = functools.lru_cache(None)(_make_row_gather)


def _nms_body(cxr, cyr, wr, hr, cxc, cyc, wc, hc, keep_ref, alive_ref):
    alive_ref[...] = jnp.ones((1, NMS_PRE), jnp.float32)
    colpos = lax.broadcasted_iota(jnp.int32, (1, NMS_PRE), 1)

    xmin_c = cxc[...] - wc[...] * 0.5
    xmax_c = cxc[...] + wc[...] * 0.5
    ymin_c = cyc[...] - hc[...] * 0.5
    ymax_c = cyc[...] + hc[...] * 0.5
    area_c = wc[...] * hc[...]

    for b in range(NBLK):
        base = b * BLK
        rs = pl.ds(base, BLK)
        cxb, wb = cxr[rs, :], wr[rs, :]
        cyb, hb = cyr[rs, :], hr[rs, :]
        xmin_r = cxb - wb * 0.5
        xmax_r = cxb + wb * 0.5
        ymin_r = cyb - hb * 0.5
        ymax_r = cyb + hb * 0.5
        area_r = wb * hb

        cs = slice(base, NMS_PRE)
        iw = jnp.clip(jnp.minimum(xmax_r, xmax_c[:, cs])
                      - jnp.maximum(xmin_r, xmin_c[:, cs]), 0.0, None)
        ih = jnp.clip(jnp.minimum(ymax_r, ymax_c[:, cs])
                      - jnp.maximum(ymin_r, ymin_c[:, cs]), 0.0, None)
        inter = iw * ih
        union = area_r + area_c[:, cs] - inter
        iou = inter / jnp.clip(union, 1e-08, None)
        rowpos = lax.broadcasted_iota(jnp.int32, (BLK, 1), 0)
        supf = ((iou > NMS_THRESH) & (colpos[:, cs] - base > rowpos)
                ).astype(jnp.float32)

        sup_blk = supf[:, :BLK]
        alive0 = alive_ref[:, base:base + BLK]

        def jacobi_cond(carry):
            _, changed, t = carry
            return changed & (t < BLK)

        def jacobi_body(carry):
            keep, _, t = carry
            s = jnp.dot(keep, sup_blk, preferred_element_type=jnp.float32)
            keep_new = alive0 * (s < 0.5).astype(jnp.float32)
            changed = jnp.any(keep_new != keep)
            return keep_new, changed, t + 1

        alive_blk, _, _ = lax.while_loop(
            jacobi_cond, jacobi_body, (alive0, jnp.bool_(True), jnp.int32(0)))
        alive_ref[:, base:base + BLK] = alive_blk

        s = jnp.dot(alive_blk, supf, preferred_element_type=jnp.float32)
        alive_ref[:, base:] = alive_ref[:, base:] * (s < 0.5).astype(jnp.float32)

    keep_ref[...] = alive_ref[...]


_nms_call = pl.pallas_call(
    _nms_body,
    out_shape=jax.ShapeDtypeStruct((1, NMS_PRE), jnp.float32),
    scratch_shapes=[
        pltpu.VMEM((1, NMS_PRE), jnp.float32),
    ],
)


def _make_finalize():
    nrows = 512
    per_w = nrows // _NW
    mesh = plsc.VectorSubcoreMesh(core_axis_name="c", subcore_axis_name="s", num_cores=_NCORES, num_subcores=_NSUB)

    @functools.partial(
        pl.kernel,
        out_type=(jax.ShapeDtypeStruct((nrows, 8), jnp.float32),
                  jax.ShapeDtypeStruct((nrows,), jnp.float32),
                  jax.ShapeDtypeStruct((nrows,), jnp.float32)),
        mesh=mesh,
        compiler_params=pltpu.CompilerParams(use_tc_tiling_on_sc=False, needs_layout_passes=False),
        scratch_types=[
            pltpu.VMEM((NMS_PRE,), jnp.float32),
            pltpu.VMEM((NMS_PRE,), jnp.int32),
            pltpu.VMEM((nrows,), jnp.int32),
            pltpu.VMEM((per_w,), jnp.int32),
            pltpu.VMEM((per_w, 8), jnp.float32),
            pltpu.VMEM((per_w,), jnp.float32),
            pltpu.VMEM((per_w,), jnp.float32),
            pltpu.SemaphoreType.DMA,
        ],
    )
    def finalize(keep_hbm, cand_hbm, table_hbm, sc_hbm, lb_hbm,
                 obox_hbm, osc_hbm, olb_hbm,
                 keep_v, cand_v, slots_v, idx_v, rows_v, s_v, l_v, sem):
        wid = lax.axis_index("s") * _NCORES + lax.axis_index("c")
        base = wid * per_w
        pltpu.sync_copy(keep_hbm, keep_v)
        pltpu.sync_copy(cand_hbm, cand_v)
        sent = jnp.full((16,), SENTINEL, jnp.int32)
        for i in range(nrows // 16):
            slots_v[pl.ds(i * 16, 16)] = sent

        def body(i, carry):
            kv = keep_v[pl.ds(i * 16, 16)]
            incl = plsc.cumsum(kv)
            ranks = (incl + carry).astype(jnp.int32) - 1
            valid = (kv > 0.5) & (ranks < NMS_POST)
            slot = jnp.where(valid, ranks, 0)
            cv = cand_v[pl.ds(i * 16, 16)]
            plsc.store_scatter(slots_v, [slot], cv, mask=valid)
            return carry + jnp.sum(kv)

        lax.fori_loop(0, NMS_PRE // 16, body, jnp.float32(0.0))

        idx_v[...] = slots_v[pl.ds(base, per_w)]
        pltpu.async_copy(table_hbm.at[idx_v], rows_v, sem).wait()
        pltpu.async_copy(sc_hbm.at[idx_v], s_v, sem).wait()
        pltpu.async_copy(lb_hbm.at[idx_v], l_v, sem).wait()
        pltpu.sync_copy(rows_v, obox_hbm.at[pl.ds(base, per_w)])
        pltpu.sync_copy(s_v, osc_hbm.at[pl.ds(base, per_w)])
        pltpu.sync_copy(l_v, olb_hbm.at[pl.ds(base, per_w)])

    return finalize


_finalize = functools.lru_cache(None)(_make_finalize)


def kernel(boxes, scores, labels):
    keys = lax.bitcast_convert_type(scores, jnp.int32)
    keys = jnp.concatenate([keys, jnp.full((NSORT - N,), -1, jnp.int32)])
    idx = jnp.arange(NSORT, dtype=jnp.int32)
    sorted_idx = _sort_call(keys.reshape(SR, SC_), idx.reshape(SR, SC_))
    cand = sorted_idx.reshape(-1)[:NMS_PRE]

    boxes8 = jnp.zeros((NPAD, 8), jnp.float32).at[:N, :7].set(boxes)
    scores_p = jnp.zeros((NPAD,), jnp.float32).at[:N].set(scores)
    labels_p = jnp.zeros((NPAD,), jnp.float32).at[:N].set(
        labels.astype(jnp.float32))

    cb = _gather_cand(NMS_PRE)(boxes8, cand)
    cx, cy, w, h = cb[:, 0], cb[:, 1], cb[:, 3], cb[:, 4]

    keep = _nms_call(cx[:, None], cy[:, None], w[:, None], h[:, None],
                     cx[None, :], cy[None, :], w[None, :], h[None, :])

    fb, fs, fl = _finalize()(keep.reshape(NMS_PRE), cand,
                             boxes8, scores_p, labels_p)
    return (fl[:NMS_POST][None, :], fs[:NMS_POST][None, :],
            fb[:NMS_POST, :7][None, :, :])
